# Initial kernel scaffold; baseline (speedup 1.0000x reference)
import jax
import jax.numpy as jnp
from jax.experimental import pallas as pl


def kernel(x, type_embeddings, atom_types, edge_index, W_gate, W_experts, b_experts):
    raise NotImplementedError("write your pallas kernel here")


# speedup vs baseline: 34.5209x; 34.5209x over previous
import jax
import jax.numpy as jnp
from jax import lax
from jax.experimental import pallas as pl
from jax.experimental.pallas import tpu as pltpu
from jax.experimental.pallas import tpu_sc as plsc

NUM_IN = 64
NUM_OUT = 64
N_EXPERTS = 8
TOP_K = 2
TEBD_DIM = 64
NTYPES = 128
NB = 1
NLOC = 8192
N_EDGE = 524288

_NC = 2
_NS = 16
_NW = _NC * _NS
_STREAMS = (N_EDGE // 16, 3 * N_EDGE // 16, 3 * N_EDGE // 4)
_SC_CHUNK = 2048

_BLK = 32768


def _router_body(tebd_ref, wg_ref, at_ref, c_atom_ref):
    logits = jnp.dot(wg_ref[...], tebd_ref[...], preferred_element_type=jnp.float32)
    eidx = lax.broadcasted_iota(jnp.int32, (N_EXPERTS, NTYPES), 0)
    m1 = jnp.max(logits, axis=0, keepdims=True)
    i1 = jnp.min(jnp.where(logits == m1, eidx, N_EXPERTS), axis=0, keepdims=True)
    masked = jnp.where(eidx == i1, -jnp.inf, logits)
    m2 = jnp.max(masked, axis=0, keepdims=True)
    i2 = jnp.min(jnp.where(masked == m2, eidx, N_EXPERTS), axis=0, keepdims=True)
    e2 = jnp.exp(m2 - m1)
    denom = 1.0 + e2
    w1 = 1.0 / denom
    w2 = e2 / denom
    ctab_t = jnp.where(eidx == i1, w1, 0.0) + jnp.where(eidx == i2, w2, 0.0)
    tidx = lax.broadcasted_iota(jnp.int32, (NTYPES, NLOC), 0)
    onehot_t = (tidx == at_ref[...]).astype(jnp.float32)
    c_atom_ref[...] = jnp.dot(ctab_t, onehot_t,
                              preferred_element_type=jnp.float32)


def _gather_body(tab_hbm, eidx_hbm, out_hbm, tab_v, idx_v, out_v, tab_sh, *,
                 start, count, ch):
    sid = lax.axis_index("s")
    wid = sid * _NC + lax.axis_index("c")

    @pl.when(sid == 0)
    def _():
        pltpu.sync_copy(tab_hbm, tab_sh)
    plsc.subcore_barrier()
    pltpu.sync_copy(tab_sh, tab_v)
    row16 = [jnp.full((16,), e, jnp.int32) for e in range(N_EXPERTS)]
    per_worker = count // _NW
    for k in range(per_worker // ch):
        base = wid * per_worker + k * ch
        pltpu.sync_copy(eidx_hbm.at[pl.ds(start + base, ch)], idx_v)

        @plsc.parallel_loop(0, ch // 16, unroll=8)
        def body(g):
            idx16 = idx_v[pl.ds(g * 16, 16)]
            for e in range(N_EXPERTS):
                vals = plsc.load_gather(tab_v, [row16[e], idx16])
                out_v[pl.ds(e * ch + g * 16, 16)] = vals
        for e in range(N_EXPERTS):
            pltpu.sync_copy(out_v.at[pl.ds(e * ch, ch)],
                            out_hbm.at[e, pl.ds(base, ch)])


def _moe_body(xt_ref, ct_ref, w_ref, b_ref, out_ref):
    xt = xt_ref[...]
    ct = ct_ref[...]
    xc = jnp.concatenate(
        [xt * ct[e:e + 1, :] for e in range(N_EXPERTS)], axis=0)
    xcb = xc.astype(jnp.bfloat16)
    acc = jnp.dot(b_ref[...], ct, preferred_element_type=jnp.float32)
    out_ref[...] = acc + jnp.dot(w_ref[...], xcb,
                                 preferred_element_type=jnp.float32)


def _moe_body_aliased(xt_ref, ct_ref, w_ref, b_ref, prev_ref, out_ref):
    _moe_body(xt_ref, ct_ref, w_ref, b_ref, out_ref)


def kernel(x, type_embeddings, atom_types, edge_index, W_gate, W_experts, b_experts):
    c_atom_t = pl.pallas_call(
        _router_body,
        out_shape=jax.ShapeDtypeStruct((N_EXPERTS, NLOC), jnp.float32),
    )(jnp.transpose(type_embeddings), jnp.transpose(W_gate), atom_types)

    def run_gather(start, count):
        ch = min(_SC_CHUNK, count // _NW)
        return pl.kernel(
            lambda *refs: _gather_body(*refs, start=start, count=count, ch=ch),
            out_type=jax.ShapeDtypeStruct((N_EXPERTS, count), jnp.float32),
            mesh=plsc.VectorSubcoreMesh(core_axis_name="c", subcore_axis_name="s"),
            compiler_params=pltpu.CompilerParams(needs_layout_passes=False),
            scratch_types=[
                pltpu.VMEM((N_EXPERTS, NLOC), jnp.float32),
                pltpu.VMEM((ch,), jnp.int32),
                pltpu.VMEM((N_EXPERTS * ch,), jnp.float32),
                pltpu.VMEM_SHARED((N_EXPERTS, NLOC), jnp.float32),
            ],
        )(c_atom_t, edge_index)

    w_t = jnp.transpose(W_experts, (2, 0, 1)).reshape(NUM_OUT, N_EXPERTS * NUM_IN)
    w_t = w_t.astype(jnp.bfloat16)
    b_t = jnp.transpose(b_experts)
    x_t = jnp.transpose(x)

    common = dict(
        out_shape=jax.ShapeDtypeStruct((NUM_OUT, N_EDGE), jnp.float32),
        compiler_params=pltpu.CompilerParams(
            dimension_semantics=("arbitrary",),
        ),
    )
    wb_specs = [
        pl.BlockSpec((NUM_OUT, N_EXPERTS * NUM_IN), lambda i: (0, 0)),
        pl.BlockSpec((NUM_OUT, N_EXPERTS), lambda i: (0, 0)),
    ]

    out_t = None
    start = 0
    for count in _STREAMS:
        c_h = run_gather(start, count)
        off = start // _BLK
        xspec = pl.BlockSpec((NUM_IN, _BLK), lambda i, o=off: (0, i + o))
        cspec = pl.BlockSpec((N_EXPERTS, _BLK), lambda i: (0, i))
        ospec = pl.BlockSpec((NUM_OUT, _BLK), lambda i, o=off: (0, i + o))
        if out_t is None:
            out_t = pl.pallas_call(
                _moe_body,
                grid=(count // _BLK,),
                in_specs=[xspec, cspec] + wb_specs,
                out_specs=ospec,
                **common,
            )(x_t, c_h, w_t, b_t)
        else:
            out_t = pl.pallas_call(
                _moe_body_aliased,
                grid=(count // _BLK,),
                in_specs=[xspec, cspec] + wb_specs
                + [pl.BlockSpec(memory_space=pl.ANY)],
                out_specs=ospec,
                input_output_aliases={4: 0},
                **common,
            )(x_t, c_h, w_t, b_t, out_t)
        start += count
    return jnp.transpose(out_t)
